# baseline (speedup 1.0000x reference)
import functools

import jax
import jax.numpy as jnp
from jax import lax
from jax.experimental import pallas as pl
from jax.experimental.pallas import tpu as pltpu
from jax.experimental.pallas import tpu_sc as plsc

ROWS, D, F = 8192, 512, 4
NC, NS = 2, 16
NW = NC * NS
RPW = ROWS // NW
CH = 64
NCHUNK = RPW // CH

_mesh = plsc.VectorSubcoreMesh(core_axis_name="c", subcore_axis_name="s")


@functools.partial(
    pl.kernel,
    out_type=jax.ShapeDtypeStruct((ROWS, F, D), jnp.float32),
    mesh=_mesh,
    scratch_types=[pltpu.VMEM((CH, 1, D), jnp.float32)],
)
def _fill_encoding(x_hbm, out_hbm, buf):
    wid = lax.axis_index("s") * NC + lax.axis_index("c")
    base = wid * RPW
    for c in range(NCHUNK):
        b = base + c * CH
        pltpu.sync_copy(x_hbm.at[pl.ds(b, CH)], buf)
        for j in range(F):
            pltpu.sync_copy(buf, out_hbm.at[pl.ds(b, CH), pl.ds(j, 1)])


def kernel(x):
    out3 = _fill_encoding(x.reshape(ROWS, 1, D))
    return out3.reshape(ROWS * F, D)


# speedup vs baseline: 1.0074x; 1.0074x over previous
import functools

import jax
import jax.numpy as jnp
from jax import lax
from jax.experimental import pallas as pl
from jax.experimental.pallas import tpu as pltpu
from jax.experimental.pallas import tpu_sc as plsc

ROWS, D, F = 8192, 512, 4
NC, NS = 2, 16
NW = NC * NS
RPW = ROWS // NW
CH = 32
NCHUNK = RPW // CH
NBUF = 4

_mesh = plsc.VectorSubcoreMesh(core_axis_name="c", subcore_axis_name="s")


@functools.partial(
    pl.kernel,
    out_type=jax.ShapeDtypeStruct((ROWS, F, D), jnp.float32),
    mesh=_mesh,
    scratch_types=[
        pltpu.VMEM((NBUF, CH, 1, D), jnp.float32),
        pltpu.SemaphoreType.DMA((NBUF,)),
        pltpu.SemaphoreType.DMA((NBUF,)),
    ],
)
def _fill_encoding(x_hbm, out_hbm, buf, lsem, ssem):
    wid = lax.axis_index("s") * NC + lax.axis_index("c")
    base = wid * RPW

    def load(c):
        return pltpu.make_async_copy(
            x_hbm.at[pl.ds(base + c * CH, CH)], buf.at[c % NBUF], lsem.at[c % NBUF]
        )

    def store(c, j):
        return pltpu.make_async_copy(
            buf.at[c % NBUF],
            out_hbm.at[pl.ds(base + c * CH, CH), pl.ds(j, 1)],
            ssem.at[c % NBUF],
        )

    for c in range(min(NBUF, NCHUNK)):
        load(c).start()
    inflight = {}
    for c in range(NCHUNK):
        load(c).wait()
        stores = [store(c, j) for j in range(F)]
        for s in stores:
            s.start()
        inflight[c] = stores
        if c - 1 in inflight:
            for s in inflight.pop(c - 1):
                s.wait()
            if c - 1 + NBUF < NCHUNK:
                load(c - 1 + NBUF).start()
    for stores in inflight.values():
        for s in stores:
            s.wait()


def kernel(x):
    out3 = _fill_encoding(x.reshape(ROWS, 1, D))
    return out3.reshape(ROWS * F, D)


# speedup vs baseline: 2.7395x; 2.7194x over previous
import functools

import jax
import jax.numpy as jnp
from jax import lax
from jax.experimental import pallas as pl
from jax.experimental.pallas import tpu as pltpu
from jax.experimental.pallas import tpu_sc as plsc

ROWS, D, F = 8192, 512, 4
NC, NS = 2, 16
NW = NC * NS
RPW = ROWS // NW
CH = 64
CHG = CH // 8
NCHUNK = RPW // CH
NBUF = 2

_mesh = plsc.VectorSubcoreMesh(core_axis_name="c", subcore_axis_name="s")


@functools.partial(
    pl.kernel,
    out_type=jax.ShapeDtypeStruct((ROWS * F, D), jnp.float32),
    mesh=_mesh,
    scratch_types=[
        pltpu.VMEM((NBUF, CHG, 8, D), jnp.float32),
        pltpu.SemaphoreType.DMA((NBUF,)),
        pltpu.SemaphoreType.DMA((NBUF,)),
    ],
)
def _fill_encoding(x_hbm, out_hbm, buf, lsem, ssem):
    wid = lax.axis_index("s") * NC + lax.axis_index("c")
    base = wid * RPW
    x3 = x_hbm.reshape(ROWS // 8, 8, D)
    o5 = out_hbm.reshape(ROWS // 8, 8 * F, D)

    def load(c):
        a0 = (base + c * CH) // 8
        return pltpu.make_async_copy(
            x3.at[pl.ds(a0, CHG)], buf.at[c % NBUF], lsem.at[c % NBUF]
        )

    def store(c, r, k):
        g0 = (base + c * CH) // 8
        src = buf.at[c % NBUF, :, pl.ds(r, 1), :]
        dst = o5.at[pl.ds(g0, CHG), pl.ds(F * r + k, 1), :]
        return pltpu.make_async_copy(src, dst, ssem.at[c % NBUF])

    for c in range(min(NBUF, NCHUNK)):
        load(c).start()
    for c in range(NCHUNK):
        load(c).wait()

        def fire(i, _, c=c):
            store(c, i // F, lax.rem(i, F)).start()
            return 0

        def drain(i, _, c=c):
            store(c, 0, 0).wait()
            return 0

        lax.fori_loop(0, 8 * F, fire, 0)
        lax.fori_loop(0, 8 * F, drain, 0)
        if c + NBUF < NCHUNK:
            load(c + NBUF).start()


def kernel(x):
    return _fill_encoding(x)
